# baseline (speedup 1.0000x reference)
import jax
import jax.numpy as jnp
from jax import lax
from jax.experimental import pallas as pl
from jax.experimental.pallas import tpu as pltpu
from jax.experimental.pallas import tpu_sc as plsc

_IN = 1000000
_B = 4096
_L = 50
_D = 128
_NLANE = 16
_NVREG = _D // _NLANE

_NC = 2
_NS = 16
_NW = _NC * _NS
_NCHUNK = 1
_CB = _B // _NCHUNK
_RPW = _CB // _NW
_BLK = 4
_IDXB = _BLK * _L
_NBLK = _RPW // _BLK
_SPLITS = ((0, 56), (56, 48), (104, 48), (152, 48))


def _make_sc_body(row0):
    def body(xf_hbm, table_hbm, out_hbm, idx_v, buf_v, out_v,
             sem0, sem1, sem2, sem3):
        c = lax.axis_index("c")
        s = lax.axis_index("s")
        wid = s * _NC + c
        rbase = wid * _RPW

        pltpu.sync_copy(xf_hbm.at[pl.ds((row0 + rbase) * _L, _RPW * _L)],
                        idx_v)

        sems = (sem0, sem1, sem2, sem3)

        def start(b):
            p = b % 4
            for off, ln in _SPLITS:
                pltpu.async_copy(
                    table_hbm.at[idx_v.at[pl.ds(b * _IDXB + off, ln)]],
                    buf_v.at[p, pl.ds(off, ln)],
                    sems[p],
                )

        def wait(b):
            p = b % 4
            for off, ln in _SPLITS:
                pltpu.make_async_copy(
                    table_hbm.at[idx_v.at[pl.ds(b * _IDXB + off, ln)]],
                    buf_v.at[p, pl.ds(off, ln)],
                    sems[p],
                ).wait()

        for b in range(4):
            start(b)
        for b in range(_NBLK):
            wait(b)
            p = b % 4

            def row_body(r, _, p=p, b=b):
                base = r * _L

                def acc_body(i, accs):
                    row = base + i
                    return tuple(
                        accs[v] + buf_v[p, row, pl.ds(v * _NLANE, _NLANE)]
                        for v in range(_NVREG)
                    )

                accs = lax.fori_loop(
                    0, _L, acc_body,
                    tuple(jnp.zeros((_NLANE,), jnp.float32)
                          for _ in range(_NVREG)),
                )
                out_row = b * _BLK + r
                for v in range(_NVREG):
                    out_v[out_row, pl.ds(v * _NLANE, _NLANE)] = accs[v]
                return 0

            lax.fori_loop(0, _BLK, row_body, 0)
            if b + 4 < _NBLK:
                start(b + 4)

        pltpu.sync_copy(out_v, out_hbm.at[pl.ds(rbase, _RPW)])

    return body


def _sc_gather_sum(xf, table, chunk):
    mesh = plsc.VectorSubcoreMesh(core_axis_name="c", subcore_axis_name="s")
    return pl.kernel(
        _make_sc_body(chunk * _CB),
        out_type=jax.ShapeDtypeStruct((_CB, _D), jnp.float32),
        mesh=mesh,
        scratch_types=[
            pltpu.VMEM((_RPW * _L,), jnp.int32),
            pltpu.VMEM((4, _IDXB, _D), jnp.float32),
            pltpu.VMEM((_RPW, _D), jnp.float32),
            pltpu.SemaphoreType.DMA,
            pltpu.SemaphoreType.DMA,
            pltpu.SemaphoreType.DMA,
            pltpu.SemaphoreType.DMA,
        ],
    )(xf, table)


_TC_ROWS = 1024
_NSTEP = _CB // _TC_ROWS


def _tc_body(*refs):
    sum_ref, x_ref, bias_ref, w_ref, b_ref = refs[:5]
    o_ref = refs[-1]
    sizes = jnp.sum((x_ref[...] != _IN).astype(jnp.float32), axis=1,
                    keepdims=True)
    q = jnp.maximum(sum_ref[...] / sizes + bias_ref[...][None, :], 0.0)
    o_ref[...] = lax.dot_general(
        w_ref[...], q, (((1,), (1,)), ((), ())),
        precision=lax.Precision.DEFAULT,
        preferred_element_type=jnp.float32,
    ) + b_ref[...][:, None]


def _tc_tail(summed, x, bias, w, b_lin, chunk, prev):
    nout = w.shape[0]
    in_specs = [
        pl.BlockSpec((_TC_ROWS, _D), lambda i: (i, 0)),
        pl.BlockSpec((_TC_ROWS, _L),
                     lambda i, c=chunk: (i + c * _NSTEP, 0)),
        pl.BlockSpec((_D,), lambda i: (0,)),
        pl.BlockSpec((nout, _D), lambda i: (0, 0)),
        pl.BlockSpec((nout,), lambda i: (0,)),
    ]
    args = [summed, x, bias, w, b_lin]
    io_alias = {}
    if prev is not None:
        in_specs.append(pl.BlockSpec(memory_space=pl.ANY))
        args.append(prev)
        io_alias = {5: 0}
    return pl.pallas_call(
        _tc_body,
        grid=(_NSTEP,),
        in_specs=in_specs,
        out_specs=pl.BlockSpec((nout, _TC_ROWS),
                               lambda i, c=chunk: (0, i + c * _NSTEP)),
        out_shape=jax.ShapeDtypeStruct((nout, _B), jnp.float32),
        input_output_aliases=io_alias,
    )(*args)


def kernel(x, table, bias, W, b_lin):
    xf = x.reshape(-1).astype(jnp.int32)
    summed = [_sc_gather_sum(xf, table, c) for c in range(_NCHUNK)]
    out_t = None
    for c in range(_NCHUNK):
        out_t = _tc_tail(summed[c], x, bias, W, b_lin, c, out_t)
    return out_t.T


# speedup vs baseline: 1.0022x; 1.0022x over previous
import jax
import jax.numpy as jnp
from jax import lax
from jax.experimental import pallas as pl
from jax.experimental.pallas import tpu as pltpu
from jax.experimental.pallas import tpu_sc as plsc

_IN = 1000000
_B = 4096
_L = 50
_D = 128
_NLANE = 16
_NVREG = _D // _NLANE

_NC = 2
_NS = 16
_NW = _NC * _NS
_NCHUNK = 1
_CB = _B // _NCHUNK
_RPW = _CB // _NW
_BLK = 4
_IDXB = _BLK * _L
_NBLK = _RPW // _BLK
_SPLITS = ((0, 104), (104, 96))


def _make_sc_body(row0):
    def body(xf_hbm, table_hbm, out_hbm, idx_v, buf_v, out_v,
             sem0, sem1, sem2, sem3):
        c = lax.axis_index("c")
        s = lax.axis_index("s")
        wid = s * _NC + c
        rbase = wid * _RPW

        pltpu.sync_copy(xf_hbm.at[pl.ds((row0 + rbase) * _L, _RPW * _L)],
                        idx_v)

        sems = (sem0, sem1, sem2, sem3)

        def start(b):
            p = b % 4
            for off, ln in _SPLITS:
                pltpu.async_copy(
                    table_hbm.at[idx_v.at[pl.ds(b * _IDXB + off, ln)]],
                    buf_v.at[p, pl.ds(off, ln)],
                    sems[p],
                )

        def wait(b):
            p = b % 4
            for off, ln in _SPLITS:
                pltpu.make_async_copy(
                    table_hbm.at[idx_v.at[pl.ds(b * _IDXB + off, ln)]],
                    buf_v.at[p, pl.ds(off, ln)],
                    sems[p],
                ).wait()

        for b in range(4):
            start(b)
        for b in range(_NBLK):
            wait(b)
            p = b % 4

            def row_body(r, _, p=p, b=b):
                base = r * _L

                def acc_body(i, accs):
                    row = base + i
                    return tuple(
                        accs[v] + buf_v[p, row, pl.ds(v * _NLANE, _NLANE)]
                        for v in range(_NVREG)
                    )

                accs = lax.fori_loop(
                    0, _L, acc_body,
                    tuple(jnp.zeros((_NLANE,), jnp.float32)
                          for _ in range(_NVREG)),
                )
                out_row = b * _BLK + r
                for v in range(_NVREG):
                    out_v[out_row, pl.ds(v * _NLANE, _NLANE)] = accs[v]
                return 0

            lax.fori_loop(0, _BLK, row_body, 0)
            if b + 4 < _NBLK:
                start(b + 4)

        pltpu.sync_copy(out_v, out_hbm.at[pl.ds(rbase, _RPW)])

    return body


def _sc_gather_sum(xf, table, chunk):
    mesh = plsc.VectorSubcoreMesh(core_axis_name="c", subcore_axis_name="s")
    return pl.kernel(
        _make_sc_body(chunk * _CB),
        out_type=jax.ShapeDtypeStruct((_CB, _D), jnp.float32),
        mesh=mesh,
        scratch_types=[
            pltpu.VMEM((_RPW * _L,), jnp.int32),
            pltpu.VMEM((4, _IDXB, _D), jnp.float32),
            pltpu.VMEM((_RPW, _D), jnp.float32),
            pltpu.SemaphoreType.DMA,
            pltpu.SemaphoreType.DMA,
            pltpu.SemaphoreType.DMA,
            pltpu.SemaphoreType.DMA,
        ],
    )(xf, table)


_TC_ROWS = 1024
_NSTEP = _CB // _TC_ROWS


def _tc_body(*refs):
    sum_ref, x_ref, bias_ref, w_ref, b_ref = refs[:5]
    o_ref = refs[-1]
    sizes = jnp.sum((x_ref[...] != _IN).astype(jnp.float32), axis=1,
                    keepdims=True)
    q = jnp.maximum(sum_ref[...] / sizes + bias_ref[...][None, :], 0.0)
    o_ref[...] = lax.dot_general(
        w_ref[...], q, (((1,), (1,)), ((), ())),
        precision=lax.Precision.DEFAULT,
        preferred_element_type=jnp.float32,
    ) + b_ref[...][:, None]


def _tc_tail(summed, x, bias, w, b_lin, chunk, prev):
    nout = w.shape[0]
    in_specs = [
        pl.BlockSpec((_TC_ROWS, _D), lambda i: (i, 0)),
        pl.BlockSpec((_TC_ROWS, _L),
                     lambda i, c=chunk: (i + c * _NSTEP, 0)),
        pl.BlockSpec((_D,), lambda i: (0,)),
        pl.BlockSpec((nout, _D), lambda i: (0, 0)),
        pl.BlockSpec((nout,), lambda i: (0,)),
    ]
    args = [summed, x, bias, w, b_lin]
    io_alias = {}
    if prev is not None:
        in_specs.append(pl.BlockSpec(memory_space=pl.ANY))
        args.append(prev)
        io_alias = {5: 0}
    return pl.pallas_call(
        _tc_body,
        grid=(_NSTEP,),
        in_specs=in_specs,
        out_specs=pl.BlockSpec((nout, _TC_ROWS),
                               lambda i, c=chunk: (0, i + c * _NSTEP)),
        out_shape=jax.ShapeDtypeStruct((nout, _B), jnp.float32),
        input_output_aliases=io_alias,
    )(*args)


def kernel(x, table, bias, W, b_lin):
    xf = x.reshape(-1).astype(jnp.int32)
    summed = [_sc_gather_sum(xf, table, c) for c in range(_NCHUNK)]
    out_t = None
    for c in range(_NCHUNK):
        out_t = _tc_tail(summed[c], x, bias, W, b_lin, c, out_t)
    return out_t.T
